# baseline (speedup 1.0000x reference)
import functools

import jax
import jax.numpy as jnp
import numpy as np
from jax import lax
from jax.experimental import pallas as pl
from jax.experimental.pallas import tpu as pltpu
from jax.experimental.pallas import tpu_sc as plsc

B, T, K, D = 8, 576, 512, 64
N = B * T
NIDX = 2 * N
DIMS0 = (((0,), (0,)), ((), ()))


def _aug_ct(ct):
    c2 = jnp.sum(ct * ct, axis=0, keepdims=True)
    return jnp.concatenate([-2.0 * ct, c2], axis=0)


def _aug_z(ab):
    ones_row = jnp.ones((1, T), jnp.float32)
    return jnp.concatenate([ab, ones_row], axis=0)


def _split(x):
    hi = x.astype(jnp.bfloat16)
    lo = (x - hi.astype(jnp.float32)).astype(jnp.bfloat16)
    return hi, lo


def _topk_body(zbd_ref, ct_ref, ifl_ref, pair_ref):
    ct = ct_ref[...]
    ctaug = _aug_ct(ct)
    ch, cl = _split(ctaug)
    iota_f = lax.broadcasted_iota(jnp.int32, (T, K), 1).astype(jnp.float32)
    for b in range(B):
        ab = zbd_ref[b * D:(b + 1) * D, :]
        zaug = _aug_z(ab)
        zh, zl = _split(zaug)
        cross = (lax.dot_general(zh, ch, DIMS0, preferred_element_type=jnp.float32)
                 + (lax.dot_general(zh, cl, DIMS0, preferred_element_type=jnp.float32)
                    + lax.dot_general(zl, ch, DIMS0, preferred_element_type=jnp.float32)))
        z2 = jnp.transpose(jnp.sum(ab * ab, axis=0, keepdims=True))
        dist = z2 + cross
        m1 = jnp.min(dist, axis=1, keepdims=True)
        a1f = jnp.min(jnp.where(dist == m1, iota_f, np.float32(K)),
                      axis=1, keepdims=True)
        dm = jnp.where(iota_f == a1f, jnp.inf, dist)
        m2 = jnp.min(dm, axis=1, keepdims=True)
        a2f = jnp.min(jnp.where(dm == m2, iota_f, np.float32(K)),
                      axis=1, keepdims=True)
        both = jnp.concatenate([a1f, a2f], axis=1)
        bi = jnp.transpose(both).astype(jnp.int32)
        ifl_ref[0:1, pl.ds(b * T, T)] = bi[0:1, :]
        ifl_ref[0:1, pl.ds(N + b * T, T)] = bi[1:2, :]
    c_rows = jnp.transpose(ct)
    pair_ref[...] = jnp.concatenate([c_rows, c_rows], axis=1)


_topk_call = pl.pallas_call(
    _topk_body,
    out_shape=(
        jax.ShapeDtypeStruct((1, NIDX), jnp.int32),
        jax.ShapeDtypeStruct((K, 2 * D), jnp.float32),
    ),
)


@functools.cache
def _gather_call():
    info = plsc.get_sparse_core_info()
    nw = info.num_cores * info.num_subcores
    rows_per_w = NIDX // nw
    mesh = plsc.VectorSubcoreMesh(core_axis_name="c", subcore_axis_name="s")

    @functools.partial(
        pl.kernel,
        mesh=mesh,
        out_type=jax.ShapeDtypeStruct((NIDX, 2 * D), jnp.float32),
        scratch_types=[
            pltpu.VMEM((rows_per_w,), jnp.int32),
            pltpu.VMEM((rows_per_w, 2 * D), jnp.float32),
            pltpu.SemaphoreType.DMA,
        ],
    )
    def gather_k(table_hbm, idx_hbm, out_hbm, idx_v, rows_v, sem):
        wid = lax.axis_index("s") * info.num_cores + lax.axis_index("c")
        base = wid * rows_per_w
        pltpu.sync_copy(idx_hbm.at[pl.ds(base, rows_per_w)], idx_v)
        pltpu.async_copy(table_hbm.at[idx_v], rows_v, sem).wait()
        pltpu.sync_copy(rows_v, out_hbm.at[pl.ds(base, rows_per_w)])

    return gather_k


def _dist_body(zbd_ref, ct_ref, dist_ref, ze_ref, ct_out_ref):
    ct = ct_ref[...]
    ctaug = _aug_ct(ct).astype(jnp.bfloat16)
    for b in range(B):
        ab = zbd_ref[b * D:(b + 1) * D, :]
        cross = lax.dot_general(_aug_z(ab).astype(jnp.bfloat16), ctaug,
                                DIMS0, preferred_element_type=jnp.float32)
        z2 = jnp.transpose(jnp.sum(ab * ab, axis=0, keepdims=True))
        dist_ref[b, :, :] = z2 + cross
        ze_ref[b, :, :] = ab
    ct_out_ref[...] = ct


_dist_call = pl.pallas_call(
    _dist_body,
    out_shape=(
        jax.ShapeDtypeStruct((B, T, K), jnp.float32),
        jax.ShapeDtypeStruct((B, D, T), jnp.float32),
        jax.ShapeDtypeStruct((D, K), jnp.float32),
    ),
)


def _ref_order_dist(zt, rt):
    diff = zt - rt
    prod = (diff * diff).reshape(8, 8, T)
    a = prod[:, 0:4, :] + prod[:, 4:8, :]
    b = a[:, 0:2, :] + a[:, 2:4, :]
    g = (b[:, 0:1, :] + b[:, 1:2, :]).reshape(8, T)
    s = g[0:1, :]
    for j in range(1, 8):
        s = s + g[j:j + 1, :]
    return s


def _refine_body(zbd_ref, rows_ref, idx_ref,
                 q_ref, ek_ref, zq_ref, perp_ref):
    iota_kr = lax.broadcasted_iota(jnp.int32, (K, T), 0)
    counts = jnp.zeros((K, 1), jnp.float32)
    for b in range(B):
        ab = zbd_ref[b * D:(b + 1) * D, :]
        r1t = jnp.transpose(rows_ref[b * T:(b + 1) * T, 0:D])
        r2t = jnp.transpose(rows_ref[N + b * T:N + (b + 1) * T, 0:D])
        d1 = _ref_order_dist(ab, r1t)
        d2 = _ref_order_dist(ab, r2t)
        a1 = idx_ref[0:1, b * T:(b + 1) * T]
        a2 = idx_ref[0:1, N + b * T:N + (b + 1) * T]
        pick2 = (d2 < d1) | ((d2 == d1) & (a2 < a1))
        q = jnp.where(pick2, a2, a1)
        q_ref[b:b + 1, :] = q
        ek = jnp.where(pick2, r2t, r1t)
        ek_ref[b, :, :] = ek
        zq_ref[b, :, :] = ab + (ek - ab)
        onehot = (iota_kr == q).astype(jnp.float32)
        counts = counts + jnp.sum(onehot, axis=1, keepdims=True)
    avg = counts / np.float32(N)
    ent = jnp.sum(avg * jnp.log(avg + 1e-10), axis=0, keepdims=True)
    perp_ref[...] = jnp.exp(-ent)


_refine_call = pl.pallas_call(
    _refine_body,
    out_shape=(
        jax.ShapeDtypeStruct((B, T), jnp.int32),
        jax.ShapeDtypeStruct((B, D, T), jnp.float32),
        jax.ShapeDtypeStruct((B, D, T), jnp.float32),
        jax.ShapeDtypeStruct((1, 1), jnp.float32),
    ),
)


def kernel(z_e, codebook):
    zbd = jnp.transpose(z_e, (0, 2, 1)).reshape(B * D, T)
    ct = jnp.transpose(codebook)

    idxflat, pairtab = _topk_call(zbd, ct)
    rows = _gather_call()(pairtab, idxflat.reshape(NIDX))
    dist, ze3, ct_o = _dist_call(zbd, ct)
    q, ek3, zq3, perp = _refine_call(zbd, rows, idxflat)

    return (jnp.transpose(ze3, (0, 2, 1)),
            jnp.transpose(ct_o),
            dist,
            q,
            perp.reshape(()),
            jnp.transpose(ek3, (0, 2, 1)),
            jnp.transpose(zq3, (0, 2, 1)))


# speedup vs baseline: 1.0323x; 1.0323x over previous
import functools

import jax
import jax.numpy as jnp
import numpy as np
from jax import lax
from jax.experimental import pallas as pl
from jax.experimental.pallas import tpu as pltpu
from jax.experimental.pallas import tpu_sc as plsc

B, T, K, D = 8, 576, 512, 64
N = B * T
NIDX = 2 * N
DIMS0 = (((0,), (0,)), ((), ()))


def _aug_ct(ct):
    c2 = jnp.sum(ct * ct, axis=0, keepdims=True)
    return jnp.concatenate([-2.0 * ct, c2], axis=0)


def _aug_z(ab):
    ones_row = jnp.ones((1, T), jnp.float32)
    return jnp.concatenate([ab, ones_row], axis=0)


def _split(x):
    hi = x.astype(jnp.bfloat16)
    lo = (x - hi.astype(jnp.float32)).astype(jnp.bfloat16)
    return hi, lo


def _topk_body(zbd_ref, ct_ref, idx_ref, ifl_ref, pair_ref):
    ct = ct_ref[...]
    ctaug = _aug_ct(ct)
    ch, cl = _split(ctaug)
    iota_f = lax.broadcasted_iota(jnp.int32, (T, K), 1).astype(jnp.float32)
    for b in range(B):
        ab = zbd_ref[b * D:(b + 1) * D, :]
        zaug = _aug_z(ab)
        zh, zl = _split(zaug)
        cross = (lax.dot_general(zh, ch, DIMS0, preferred_element_type=jnp.float32)
                 + (lax.dot_general(zh, cl, DIMS0, preferred_element_type=jnp.float32)
                    + lax.dot_general(zl, ch, DIMS0, preferred_element_type=jnp.float32)))
        z2 = jnp.transpose(jnp.sum(ab * ab, axis=0, keepdims=True))
        dist = z2 + cross
        m1 = jnp.min(dist, axis=1, keepdims=True)
        a1f = jnp.min(jnp.where(dist == m1, iota_f, np.float32(K)),
                      axis=1, keepdims=True)
        dm = jnp.where(iota_f == a1f, jnp.inf, dist)
        m2 = jnp.min(dm, axis=1, keepdims=True)
        a2f = jnp.min(jnp.where(dm == m2, iota_f, np.float32(K)),
                      axis=1, keepdims=True)
        both = jnp.concatenate([a1f, a2f], axis=1)
        bi = jnp.transpose(both).astype(jnp.int32)
        idx_ref[:, b, :] = bi
        ifl_ref[0:1, pl.ds(b * T, T)] = bi[0:1, :]
        ifl_ref[0:1, pl.ds(N + b * T, T)] = bi[1:2, :]
    c_rows = jnp.transpose(ct)
    pair_ref[...] = jnp.concatenate([c_rows, c_rows], axis=1)


_topk_call = pl.pallas_call(
    _topk_body,
    out_shape=(
        jax.ShapeDtypeStruct((2, B, T), jnp.int32),
        jax.ShapeDtypeStruct((1, NIDX), jnp.int32),
        jax.ShapeDtypeStruct((K, 2 * D), jnp.float32),
    ),
)


@functools.cache
def _gather_call():
    info = plsc.get_sparse_core_info()
    nw = info.num_cores * info.num_subcores
    rows_per_w = NIDX // nw
    mesh = plsc.VectorSubcoreMesh(core_axis_name="c", subcore_axis_name="s")

    @functools.partial(
        pl.kernel,
        mesh=mesh,
        out_type=jax.ShapeDtypeStruct((NIDX, 2 * D), jnp.float32),
        scratch_types=[
            pltpu.VMEM((rows_per_w,), jnp.int32),
            pltpu.VMEM((rows_per_w, 2 * D), jnp.float32),
            pltpu.SemaphoreType.DMA,
        ],
    )
    def gather_k(table_hbm, idx_hbm, out_hbm, idx_v, rows_v, sem):
        wid = lax.axis_index("s") * info.num_cores + lax.axis_index("c")
        base = wid * rows_per_w
        pltpu.sync_copy(idx_hbm.at[pl.ds(base, rows_per_w)], idx_v)
        pltpu.async_copy(table_hbm.at[idx_v], rows_v, sem).wait()
        pltpu.sync_copy(rows_v, out_hbm.at[pl.ds(base, rows_per_w)])

    return gather_k


def _dist_body(zbd_ref, ct_ref, dist_ref, ze_ref, ct_out_ref):
    ct = ct_ref[...]
    ctaug = _aug_ct(ct).astype(jnp.bfloat16)
    for b in range(B):
        ab = zbd_ref[b * D:(b + 1) * D, :]
        cross = lax.dot_general(_aug_z(ab).astype(jnp.bfloat16), ctaug,
                                DIMS0, preferred_element_type=jnp.float32)
        z2 = jnp.transpose(jnp.sum(ab * ab, axis=0, keepdims=True))
        dist_ref[b, :, :] = z2 + cross
        ze_ref[b, :, :] = ab
    ct_out_ref[...] = ct


_dist_call = pl.pallas_call(
    _dist_body,
    out_shape=(
        jax.ShapeDtypeStruct((B, T, K), jnp.float32),
        jax.ShapeDtypeStruct((B, D, T), jnp.float32),
        jax.ShapeDtypeStruct((D, K), jnp.float32),
    ),
)


def _ref_order_dist(zt, rt):
    diff = zt - rt
    prod = (diff * diff).reshape(8, 8, T)
    a = prod[:, 0:4, :] + prod[:, 4:8, :]
    b = a[:, 0:2, :] + a[:, 2:4, :]
    g = (b[:, 0:1, :] + b[:, 1:2, :]).reshape(8, T)
    s = g[0:1, :]
    for j in range(1, 8):
        s = s + g[j:j + 1, :]
    return s


def _refine_body(zbd_ref, rows_ref, idx_ref,
                 q_ref, ek_ref, zq_ref, perp_ref):
    iota_kr = lax.broadcasted_iota(jnp.int32, (K, T), 0)
    counts = jnp.zeros((K, 1), jnp.float32)
    for b in range(B):
        ab = zbd_ref[b * D:(b + 1) * D, :]
        r1t = jnp.transpose(rows_ref[b * T:(b + 1) * T, 0:D])
        r2t = jnp.transpose(rows_ref[N + b * T:N + (b + 1) * T, 0:D])
        d1 = _ref_order_dist(ab, r1t)
        d2 = _ref_order_dist(ab, r2t)
        a1 = idx_ref[0:1, b, :]
        a2 = idx_ref[1:2, b, :]
        pick2 = (d2 < d1) | ((d2 == d1) & (a2 < a1))
        q = jnp.where(pick2, a2, a1)
        q_ref[b:b + 1, :] = q
        ek = jnp.where(pick2, r2t, r1t)
        ek_ref[b, :, :] = ek
        zq_ref[b, :, :] = ab + (ek - ab)
        onehot = (iota_kr == q).astype(jnp.float32)
        counts = counts + jnp.sum(onehot, axis=1, keepdims=True)
    avg = counts / np.float32(N)
    ent = jnp.sum(avg * jnp.log(avg + 1e-10), axis=0, keepdims=True)
    perp_ref[...] = jnp.exp(-ent)


_refine_call = pl.pallas_call(
    _refine_body,
    out_shape=(
        jax.ShapeDtypeStruct((B, T), jnp.int32),
        jax.ShapeDtypeStruct((B, D, T), jnp.float32),
        jax.ShapeDtypeStruct((B, D, T), jnp.float32),
        jax.ShapeDtypeStruct((1, 1), jnp.float32),
    ),
)


def kernel(z_e, codebook):
    zbd = jnp.transpose(z_e, (0, 2, 1)).reshape(B * D, T)
    ct = jnp.transpose(codebook)

    idx3, idxflat, pairtab = _topk_call(zbd, ct)
    rows = _gather_call()(pairtab, idxflat.reshape(NIDX))
    dist, ze3, ct_o = _dist_call(zbd, ct)
    q, ek3, zq3, perp = _refine_call(zbd, rows, idx3)

    return (jnp.transpose(ze3, (0, 2, 1)),
            jnp.transpose(ct_o),
            dist,
            q,
            perp.reshape(()),
            jnp.transpose(ek3, (0, 2, 1)),
            jnp.transpose(zq3, (0, 2, 1)))
